# baseline (speedup 1.0000x reference)
import functools

import jax
import jax.numpy as jnp
from jax import lax
from jax.experimental import pallas as pl
from jax.experimental.pallas import tpu as pltpu
from jax.experimental.pallas import tpu_sc as plsc

BATCH = 16384
EMBED = 64
NC = 2
NS = 16
NW = NC * NS
B_PER_W = BATCH // NW
CHUNK = 128
N_CHUNKS = B_PER_W // CHUNK
PAIR = 4096


def _pack_body(ta_ref, tb_ref, out_ref):
  out_ref[...] = jnp.concatenate([ta_ref[...], tb_ref[...]], axis=0).T


def _pack(tab_t):
  n = tab_t.shape[1]
  nb = (n + 2 * PAIR - 1) // (2 * PAIR)
  last = (n + PAIR - 1) // PAIR - 1
  return pl.pallas_call(
      _pack_body,
      grid=(nb,),
      in_specs=[
          pl.BlockSpec((EMBED, PAIR), lambda m: (0, 2 * m)),
          pl.BlockSpec((EMBED, PAIR),
                       lambda m: (0, jnp.minimum(2 * m + 1, last))),
      ],
      out_specs=pl.BlockSpec((PAIR, 128), lambda m: (m, 0)),
      out_shape=jax.ShapeDtypeStruct((nb * PAIR, 128), jnp.float32),
  )(tab_t, tab_t)


def _sc_gather(q_ids, pair):
  mesh = plsc.VectorSubcoreMesh(core_axis_name="c", subcore_axis_name="s")

  @functools.partial(
      pl.kernel,
      out_type=jax.ShapeDtypeStruct((BATCH, 128), jnp.float32),
      mesh=mesh,
      scratch_types=[
          pltpu.VMEM((B_PER_W,), jnp.int32),
          pltpu.VMEM((B_PER_W, 128), jnp.float32),
          pltpu.SemaphoreType.DMA,
      ],
  )
  def k(ids_hbm, pair_hbm, out, idx, rows, sem):
    wid = lax.axis_index("s") * NC + lax.axis_index("c")
    base = wid * B_PER_W
    pltpu.sync_copy(ids_hbm.at[pl.ds(base, B_PER_W)], idx)
    copies = []
    for j in range(B_PER_W // 16):
      iv = idx[pl.ds(j * 16, 16)]
      copies.append(pltpu.async_copy(
          pair_hbm.at[iv], rows.at[pl.ds(j * 16, 16)], sem))
    for c in copies:
      c.wait()
    pltpu.sync_copy(rows, out.at[pl.ds(base, B_PER_W)])

  return k(q_ids, pair)


def _mlp_body(u, i, hu, hi, w1u, w1i, b1, w2, b2, w3, b3, out):
  f32 = jnp.float32
  hp = jax.lax.Precision.HIGHEST
  xu = jnp.where(hu[...] > 0.5, u[...][:, EMBED:], u[...][:, :EMBED])
  xi = jnp.where(hi[...] > 0.5, i[...][:, EMBED:], i[...][:, :EMBED])
  h = (jnp.dot(xu, w1u[...], preferred_element_type=f32, precision=hp)
       + jnp.dot(xi, w1i[...], preferred_element_type=f32, precision=hp)
       + b1[...])
  h = jnp.maximum(h, 0.0)
  h2 = jnp.dot(h, w2[...], preferred_element_type=f32, precision=hp) + b2[...]
  h2 = jnp.maximum(h2, 0.0)
  out[...] = jnp.dot(h2, w3[...], preferred_element_type=f32,
                     precision=hp) + b3[...]


def _mlp(u_pr, i_pr, hu, hi, W1u, W1i, b1, W2, b2, W3, b3, bm=2048):
  grid = (BATCH // bm,)
  full = lambda shape: pl.BlockSpec(shape, lambda m: (0,) * len(shape))
  return pl.pallas_call(
      _mlp_body,
      grid=grid,
      in_specs=[
          pl.BlockSpec((bm, 128), lambda m: (m, 0)),
          pl.BlockSpec((bm, 128), lambda m: (m, 0)),
          pl.BlockSpec((bm, 1), lambda m: (m, 0)),
          pl.BlockSpec((bm, 1), lambda m: (m, 0)),
          full((EMBED, 128)),
          full((EMBED, 128)),
          full((1, 128)),
          full((128, 64)),
          full((1, 64)),
          full((EMBED, 1)),
          full((1, 1)),
      ],
      out_specs=pl.BlockSpec((bm, 1), lambda m: (m, 0)),
      out_shape=jax.ShapeDtypeStruct((BATCH, 1), jnp.float32),
  )(u_pr, i_pr, hu, hi, W1u, W1i, b1, W2, b2, W3, b3)


def kernel(user_ids, item_ids, user_table, item_table, W1, b1, W2, b2, W3, b3):
  uid = user_ids.astype(jnp.int32)
  iid = item_ids.astype(jnp.int32)
  qu = (uid // (2 * PAIR)) * PAIR + uid % PAIR
  qi = (iid // (2 * PAIR)) * PAIR + iid % PAIR
  hu = ((uid // PAIR) % 2).astype(jnp.float32).reshape(-1, 1)
  hi = ((iid // PAIR) % 2).astype(jnp.float32).reshape(-1, 1)
  ipair = _pack(item_table.T)
  i_pr = _sc_gather(qi, ipair)
  upair = _pack(user_table.T)
  u_pr = _sc_gather(qu, upair)
  return _mlp(u_pr, i_pr, hu, hi,
              W1[:EMBED], W1[EMBED:], b1.reshape(1, -1),
              W2, b2.reshape(1, -1), W3, b3.reshape(1, 1))


# speedup vs baseline: 1.2287x; 1.2287x over previous
import functools

import jax
import jax.numpy as jnp
from jax import lax
from jax.experimental import pallas as pl
from jax.experimental.pallas import tpu as pltpu
from jax.experimental.pallas import tpu_sc as plsc

BATCH = 16384
EMBED = 64
NC = 2
NS = 16
NW = NC * NS
B_PER_W = BATCH // NW
PAIR = 4096


def _bf16_bits(x_f32_i32):
  u = x_f32_i32
  bias = jnp.int32(0x7FFF) + (lax.shift_right_logical(u, 16) & 1)
  return lax.shift_right_logical(u + bias, 16)


def _pack_body(ta_ref, tb_ref, tc_ref, td_ref, out_ref):
  i32 = jnp.int32
  t1 = jnp.concatenate([ta_ref[...], tb_ref[...]], axis=0).T
  t2 = jnp.concatenate([tc_ref[...], td_ref[...]], axis=0).T
  lo = _bf16_bits(lax.bitcast_convert_type(t1, i32))
  hi = _bf16_bits(lax.bitcast_convert_type(t2, i32))
  out_ref[...] = lo | lax.shift_left(hi, 16)


def _pack(tab_t):
  n = tab_t.shape[1]
  nb = (n + 4 * PAIR - 1) // (4 * PAIR)
  last = (n + PAIR - 1) // PAIR - 1
  spec = lambda t: pl.BlockSpec(
      (EMBED, PAIR), lambda m, t=t: (0, jnp.minimum(4 * m + t, last)))
  return pl.pallas_call(
      _pack_body,
      grid=(nb,),
      in_specs=[spec(0), spec(1), spec(2), spec(3)],
      out_specs=pl.BlockSpec((PAIR, 128), lambda m: (m, 0)),
      out_shape=jax.ShapeDtypeStruct((nb * PAIR, 128), jnp.int32),
  )(tab_t, tab_t, tab_t, tab_t)


def _sc_gather(q_ids, quad):
  mesh = plsc.VectorSubcoreMesh(core_axis_name="c", subcore_axis_name="s")

  @functools.partial(
      pl.kernel,
      out_type=jax.ShapeDtypeStruct((BATCH, 128), jnp.int32),
      mesh=mesh,
      scratch_types=[
          pltpu.VMEM((B_PER_W,), jnp.int32),
          pltpu.VMEM((B_PER_W, 128), jnp.int32),
          pltpu.SemaphoreType.DMA,
      ],
  )
  def k(ids_hbm, quad_hbm, out, idx, rows, sem):
    wid = lax.axis_index("s") * NC + lax.axis_index("c")
    base = wid * B_PER_W
    pltpu.sync_copy(ids_hbm.at[pl.ds(base, B_PER_W)], idx)
    copies = []
    for j in range(B_PER_W // 16):
      iv = idx[pl.ds(j * 16, 16)]
      copies.append(pltpu.async_copy(
          quad_hbm.at[iv], rows.at[pl.ds(j * 16, 16)], sem))
    for c in copies:
      c.wait()
    pltpu.sync_copy(rows, out.at[pl.ds(base, B_PER_W)])

  return k(q_ids, quad)


def _mlp_body(u, i, su, si, w1u, w1i, b1, w2, b2, w3, b3, out):
  f32 = jnp.float32
  hp = jax.lax.Precision.HIGHEST

  def pick(quad, sel):
    s = sel[...]
    w = jnp.where(s % 2.0 > 0.5, quad[:, EMBED:], quad[:, :EMBED])
    x_lo = lax.bitcast_convert_type(lax.shift_left(w, 16), f32)
    x_hi = lax.bitcast_convert_type(w & jnp.int32(-65536), f32)
    return jnp.where(s > 1.5, x_hi, x_lo)

  xu = pick(u[...], su)
  xi = pick(i[...], si)
  h = (jnp.dot(xu, w1u[...], preferred_element_type=f32, precision=hp)
       + jnp.dot(xi, w1i[...], preferred_element_type=f32, precision=hp)
       + b1[...])
  h = jnp.maximum(h, 0.0)
  h2 = jnp.dot(h, w2[...], preferred_element_type=f32, precision=hp) + b2[...]
  h2 = jnp.maximum(h2, 0.0)
  out[...] = jnp.dot(h2, w3[...], preferred_element_type=f32,
                     precision=hp) + b3[...]


def _mlp(u_q, i_q, su, si, W1u, W1i, b1, W2, b2, W3, b3, bm=2048):
  grid = (BATCH // bm,)
  full = lambda shape: pl.BlockSpec(shape, lambda m: (0,) * len(shape))
  return pl.pallas_call(
      _mlp_body,
      grid=grid,
      in_specs=[
          pl.BlockSpec((bm, 128), lambda m: (m, 0)),
          pl.BlockSpec((bm, 128), lambda m: (m, 0)),
          pl.BlockSpec((bm, 1), lambda m: (m, 0)),
          pl.BlockSpec((bm, 1), lambda m: (m, 0)),
          full((EMBED, 128)),
          full((EMBED, 128)),
          full((1, 128)),
          full((128, 64)),
          full((1, 64)),
          full((EMBED, 1)),
          full((1, 1)),
      ],
      out_specs=pl.BlockSpec((bm, 1), lambda m: (m, 0)),
      out_shape=jax.ShapeDtypeStruct((BATCH, 1), jnp.float32),
  )(u_q, i_q, su, si, W1u, W1i, b1, W2, b2, W3, b3)


def kernel(user_ids, item_ids, user_table, item_table, W1, b1, W2, b2, W3, b3):
  uid = user_ids.astype(jnp.int32)
  iid = item_ids.astype(jnp.int32)
  qu = (uid // (4 * PAIR)) * PAIR + uid % PAIR
  qi = (iid // (4 * PAIR)) * PAIR + iid % PAIR
  su = ((uid // PAIR) % 4).astype(jnp.float32).reshape(-1, 1)
  si = ((iid // PAIR) % 4).astype(jnp.float32).reshape(-1, 1)
  ipair = _pack(item_table.T)
  i_q = _sc_gather(qi, ipair)
  upair = _pack(user_table.T)
  u_q = _sc_gather(qu, upair)
  return _mlp(u_q, i_q, su, si, W1[:EMBED], W1[EMBED:],
              b1.reshape(1, -1), W2, b2.reshape(1, -1), W3, b3.reshape(1, 1))


# speedup vs baseline: 1.4680x; 1.1948x over previous
import functools

import jax
import jax.numpy as jnp
from jax import lax
from jax.experimental import pallas as pl
from jax.experimental.pallas import tpu as pltpu
from jax.experimental.pallas import tpu_sc as plsc

BATCH = 16384
EMBED = 64
NC = 2
NS = 16
NW = NC * NS
B_PER_W = BATCH // NW
PAIR = 8192


def _bf16_bits(x_f32_i32):
  u = x_f32_i32
  bias = jnp.int32(0x7FFF) + (lax.shift_right_logical(u, 16) & 1)
  return lax.shift_right_logical(u + bias, 16)


def _pack_body(ta_ref, tb_ref, tc_ref, td_ref, out_ref):
  i32 = jnp.int32
  t1 = jnp.concatenate([ta_ref[...], tb_ref[...]], axis=0).T
  t2 = jnp.concatenate([tc_ref[...], td_ref[...]], axis=0).T
  lo = _bf16_bits(lax.bitcast_convert_type(t1, i32))
  hi = _bf16_bits(lax.bitcast_convert_type(t2, i32))
  out_ref[...] = lo | lax.shift_left(hi, 16)


def _pack(tab_t):
  n = tab_t.shape[1]
  nb = (n + 4 * PAIR - 1) // (4 * PAIR)
  last = (n + PAIR - 1) // PAIR - 1
  spec = lambda t: pl.BlockSpec(
      (EMBED, PAIR), lambda m, t=t: (0, jnp.minimum(4 * m + t, last)))
  return pl.pallas_call(
      _pack_body,
      grid=(nb,),
      in_specs=[spec(0), spec(1), spec(2), spec(3)],
      out_specs=pl.BlockSpec((PAIR, 128), lambda m: (m, 0)),
      out_shape=jax.ShapeDtypeStruct((nb * PAIR, 128), jnp.int32),
  )(tab_t, tab_t, tab_t, tab_t)


def _sc_gather(q_ids, quad):
  mesh = plsc.VectorSubcoreMesh(core_axis_name="c", subcore_axis_name="s")

  @functools.partial(
      pl.kernel,
      out_type=jax.ShapeDtypeStruct((BATCH, 128), jnp.int32),
      mesh=mesh,
      scratch_types=[
          pltpu.VMEM((B_PER_W,), jnp.int32),
          pltpu.VMEM((B_PER_W, 128), jnp.int32),
          pltpu.SemaphoreType.DMA,
      ],
  )
  def k(ids_hbm, quad_hbm, out, idx, rows, sem):
    wid = lax.axis_index("s") * NC + lax.axis_index("c")
    base = wid * B_PER_W
    pltpu.sync_copy(ids_hbm.at[pl.ds(base, B_PER_W)], idx)
    copies = []
    for j in range(B_PER_W // 16):
      iv = idx[pl.ds(j * 16, 16)]
      copies.append(pltpu.async_copy(
          quad_hbm.at[iv], rows.at[pl.ds(j * 16, 16)], sem))
    for c in copies:
      c.wait()
    pltpu.sync_copy(rows, out.at[pl.ds(base, B_PER_W)])

  return k(q_ids, quad)


def _mlp_body(u, i, su, si, w1u, w1i, b1, w2, b2, w3, b3, out):
  f32 = jnp.float32
  hp = jax.lax.Precision.DEFAULT

  def pick(quad, sel):
    s = sel[...]
    w = jnp.where(s % 2.0 > 0.5, quad[:, EMBED:], quad[:, :EMBED])
    x_lo = lax.bitcast_convert_type(lax.shift_left(w, 16), f32)
    x_hi = lax.bitcast_convert_type(w & jnp.int32(-65536), f32)
    return jnp.where(s > 1.5, x_hi, x_lo)

  xu = pick(u[...], su)
  xi = pick(i[...], si)
  h = (jnp.dot(xu, w1u[...], preferred_element_type=f32, precision=hp)
       + jnp.dot(xi, w1i[...], preferred_element_type=f32, precision=hp)
       + b1[...])
  h = jnp.maximum(h, 0.0)
  h2 = jnp.dot(h, w2[...], preferred_element_type=f32, precision=hp) + b2[...]
  h2 = jnp.maximum(h2, 0.0)
  out[...] = jnp.dot(h2, w3[...], preferred_element_type=f32,
                     precision=hp) + b3[...]


def _mlp(u_q, i_q, su, si, W1u, W1i, b1, W2, b2, W3, b3, bm=4096):
  grid = (BATCH // bm,)
  full = lambda shape: pl.BlockSpec(shape, lambda m: (0,) * len(shape))
  return pl.pallas_call(
      _mlp_body,
      grid=grid,
      in_specs=[
          pl.BlockSpec((bm, 128), lambda m: (m, 0)),
          pl.BlockSpec((bm, 128), lambda m: (m, 0)),
          pl.BlockSpec((bm, 1), lambda m: (m, 0)),
          pl.BlockSpec((bm, 1), lambda m: (m, 0)),
          full((EMBED, 128)),
          full((EMBED, 128)),
          full((1, 128)),
          full((128, 64)),
          full((1, 64)),
          full((EMBED, 1)),
          full((1, 1)),
      ],
      out_specs=pl.BlockSpec((bm, 1), lambda m: (m, 0)),
      out_shape=jax.ShapeDtypeStruct((BATCH, 1), jnp.float32),
  )(u_q, i_q, su, si, W1u, W1i, b1, W2, b2, W3, b3)


def kernel(user_ids, item_ids, user_table, item_table, W1, b1, W2, b2, W3, b3):
  uid = user_ids.astype(jnp.int32)
  iid = item_ids.astype(jnp.int32)
  qu = (uid // (4 * PAIR)) * PAIR + uid % PAIR
  qi = (iid // (4 * PAIR)) * PAIR + iid % PAIR
  su = ((uid // PAIR) % 4).astype(jnp.float32).reshape(-1, 1)
  si = ((iid // PAIR) % 4).astype(jnp.float32).reshape(-1, 1)
  ipair = _pack(item_table.T)
  i_q = _sc_gather(qi, ipair)
  upair = _pack(user_table.T)
  u_q = _sc_gather(qu, upair)
  return _mlp(u_q, i_q, su, si, W1[:EMBED], W1[EMBED:],
              b1.reshape(1, -1), W2, b2.reshape(1, -1), W3, b3.reshape(1, 1))


# speedup vs baseline: 1.5248x; 1.0387x over previous
import functools

import jax
import jax.numpy as jnp
from jax import lax
from jax.experimental import pallas as pl
from jax.experimental.pallas import tpu as pltpu
from jax.experimental.pallas import tpu_sc as plsc

BATCH = 16384
EMBED = 64
NC = 2
NS = 16
NW = NC * NS
B_PER_W = BATCH // NW
PAIR = 8192


def _bf16_bits(x_f32_i32):
  u = x_f32_i32
  bias = jnp.int32(0x7FFF) + (lax.shift_right_logical(u, 16) & 1)
  return lax.shift_right_logical(u + bias, 16)


def _pack_body(ta_ref, tb_ref, tc_ref, td_ref, out_ref):
  i32 = jnp.int32
  t1 = jnp.concatenate([ta_ref[...], tb_ref[...]], axis=0).T
  t2 = jnp.concatenate([tc_ref[...], td_ref[...]], axis=0).T
  lo = _bf16_bits(lax.bitcast_convert_type(t1, i32))
  hi = _bf16_bits(lax.bitcast_convert_type(t2, i32))
  out_ref[...] = lo | lax.shift_left(hi, 16)


def _pack(tab_t):
  n = tab_t.shape[1]
  nb = (n + 4 * PAIR - 1) // (4 * PAIR)
  last = (n + PAIR - 1) // PAIR - 1
  spec = lambda t: pl.BlockSpec(
      (EMBED, PAIR), lambda m, t=t: (0, jnp.minimum(4 * m + t, last)))
  return pl.pallas_call(
      _pack_body,
      grid=(nb,),
      in_specs=[spec(0), spec(1), spec(2), spec(3)],
      out_specs=pl.BlockSpec((PAIR, 128), lambda m: (m, 0)),
      out_shape=jax.ShapeDtypeStruct((nb * PAIR, 128), jnp.int32),
  )(tab_t, tab_t, tab_t, tab_t)


def _sc_gather(q_ids, quad):
  mesh = plsc.VectorSubcoreMesh(core_axis_name="c", subcore_axis_name="s")

  @functools.partial(
      pl.kernel,
      out_type=jax.ShapeDtypeStruct((BATCH, 128), jnp.int32),
      mesh=mesh,
      scratch_types=[
          pltpu.VMEM((B_PER_W,), jnp.int32),
          pltpu.VMEM((B_PER_W, 128), jnp.int32),
          pltpu.SemaphoreType.DMA,
      ],
  )
  def k(ids_hbm, quad_hbm, out, idx, rows, sem):
    wid = lax.axis_index("s") * NC + lax.axis_index("c")
    base = wid * B_PER_W
    pltpu.sync_copy(ids_hbm.at[pl.ds(base, B_PER_W)], idx)
    copies = []
    for j in range(B_PER_W // 16):
      iv = idx[pl.ds(j * 16, 16)]
      copies.append(pltpu.async_copy(
          quad_hbm.at[iv], rows.at[pl.ds(j * 16, 16)], sem))
    for c in copies:
      c.wait()
    pltpu.sync_copy(rows, out.at[pl.ds(base, B_PER_W)])

  return k(q_ids, quad)


def _mlp_body(u, i, su, si, w1u, w1i, b1, w2, b2, w3, b3, out):
  f32 = jnp.float32
  hp = jax.lax.Precision.DEFAULT

  ones = jnp.ones((1, EMBED), f32)

  def pick(quad, sel):
    s = jnp.dot(sel[...], ones, preferred_element_type=f32)
    w = jnp.where(s % 2.0 > 0.5, quad[:, EMBED:], quad[:, :EMBED])
    x_lo = lax.bitcast_convert_type(lax.shift_left(w, 16), f32)
    x_hi = lax.bitcast_convert_type(w & jnp.int32(-65536), f32)
    return jnp.where(s > 1.5, x_hi, x_lo)

  xu = pick(u[...], su)
  xi = pick(i[...], si)
  h = (jnp.dot(xu, w1u[...], preferred_element_type=f32, precision=hp)
       + jnp.dot(xi, w1i[...], preferred_element_type=f32, precision=hp)
       + b1[...])
  h = jnp.maximum(h, 0.0)
  h2 = jnp.dot(h, w2[...], preferred_element_type=f32, precision=hp) + b2[...]
  h2 = jnp.maximum(h2, 0.0)
  out[...] = jnp.dot(h2, w3[...], preferred_element_type=f32,
                     precision=hp) + b3[...]


def _mlp(u_q, i_q, su, si, W1u, W1i, b1, W2, b2, W3, b3, bm=4096):
  grid = (BATCH // bm,)
  full = lambda shape: pl.BlockSpec(shape, lambda m: (0,) * len(shape))
  return pl.pallas_call(
      _mlp_body,
      grid=grid,
      in_specs=[
          pl.BlockSpec((bm, 128), lambda m: (m, 0)),
          pl.BlockSpec((bm, 128), lambda m: (m, 0)),
          pl.BlockSpec((bm, 1), lambda m: (m, 0)),
          pl.BlockSpec((bm, 1), lambda m: (m, 0)),
          full((EMBED, 128)),
          full((EMBED, 128)),
          full((1, 128)),
          full((128, 64)),
          full((1, 64)),
          full((EMBED, 1)),
          full((1, 1)),
      ],
      out_specs=pl.BlockSpec((bm, 1), lambda m: (m, 0)),
      out_shape=jax.ShapeDtypeStruct((BATCH, 1), jnp.float32),
  )(u_q, i_q, su, si, W1u, W1i, b1, W2, b2, W3, b3)


def kernel(user_ids, item_ids, user_table, item_table, W1, b1, W2, b2, W3, b3):
  uid = user_ids.astype(jnp.int32)
  iid = item_ids.astype(jnp.int32)
  qu = (uid // (4 * PAIR)) * PAIR + uid % PAIR
  qi = (iid // (4 * PAIR)) * PAIR + iid % PAIR
  su = ((uid // PAIR) % 4).astype(jnp.float32).reshape(-1, 1)
  si = ((iid // PAIR) % 4).astype(jnp.float32).reshape(-1, 1)
  ipair = _pack(item_table.T)
  i_q = _sc_gather(qi, ipair)
  upair = _pack(user_table.T)
  u_q = _sc_gather(qu, upair)
  return _mlp(u_q, i_q, su, si, W1[:EMBED], W1[EMBED:],
              b1.reshape(1, -1), W2, b2.reshape(1, -1), W3, b3.reshape(1, 1))
